# baseline (device time: 165926 ns/iter reference)
import jax
import jax.numpy as jnp
from jax import lax
from jax.experimental import pallas as pl
from jax.experimental.pallas import tpu as pltpu

N_DEV = 4
SUBS = 4


def kernel(x, w_mat):
    m_per, k = x.shape
    _, n_per = w_mat.shape
    half = m_per // 2
    sub = half // SUBS

    w_mat = w_mat.astype(jnp.bfloat16)

    def body(x_ref, w_ref, out_ref, stage, xbf, commR, commL,
             sendR, recvR, sendL, recvL, copy_sems):
        my = lax.axis_index("i")
        left = (my + N_DEV - 1) % N_DEV
        right = (my + 1) % N_DEV

        barrier_sem = pltpu.get_barrier_semaphore()
        for nbr in (left, right):
            pl.semaphore_signal(
                barrier_sem, inc=1,
                device_id=(nbr,), device_id_type=pl.DeviceIdType.MESH,
            )
        pl.semaphore_wait(barrier_sem, 2)

        def rcopy(src, dst, ssem, rsem, dev):
            return pltpu.make_async_remote_copy(
                src_ref=src, dst_ref=dst, send_sem=ssem, recv_sem=rsem,
                device_id=(dev,), device_id_type=pl.DeviceIdType.MESH,
            )

        n_slabs = 2 * SUBS
        cp = pltpu.make_async_copy(
            x_ref.at[pl.ds(0, sub), :], stage.at[0], copy_sems.at[0],
        )
        cp.start()
        r_fly, l_fly = [], []
        for s in range(n_slabs):
            cp.wait()
            if s + 1 < n_slabs:
                cp = pltpu.make_async_copy(
                    x_ref.at[pl.ds((s + 1) * sub, sub), :],
                    stage.at[(s + 1) % 2],
                    copy_sems.at[(s + 1) % 2],
                )
                cp.start()
            xbf[pl.ds(s * sub, sub), :] = stage[s % 2].astype(jnp.bfloat16)
            if s < SUBS:
                c = rcopy(
                    xbf.at[pl.ds(s * sub, sub), :],
                    commR.at[0, pl.ds(s * sub, sub), :],
                    sendR.at[0, s], recvR.at[0, s], right,
                )
                c.start()
                r_fly.append(c)
            else:
                t = s - SUBS
                c = rcopy(
                    xbf.at[pl.ds(s * sub, sub), :],
                    commL.at[0, pl.ds(t * sub, sub), :],
                    sendL.at[0, t], recvL.at[0, t], left,
                )
                c.start()
                l_fly.append(c)

        def gemm_store(src, origin, row_off, rows):
            out_ref[pl.ds(origin * m_per + row_off, rows), :] = jnp.maximum(
                jnp.dot(src, w_ref[...], preferred_element_type=jnp.float32),
                0.0,
            )

        gemm_store(xbf[...], my, 0, m_per)

        for h in range(N_DEV - 2):
            r_next, l_next = [], []
            for s in range(SUBS):
                r_fly[s].wait()
                c = rcopy(
                    commR.at[h, pl.ds(s * sub, sub), :],
                    commR.at[h + 1, pl.ds(s * sub, sub), :],
                    sendR.at[h + 1, s], recvR.at[h + 1, s], right,
                )
                c.start()
                r_next.append(c)
                l_fly[s].wait()
                c = rcopy(
                    commL.at[h, pl.ds(s * sub, sub), :],
                    commL.at[h + 1, pl.ds(s * sub, sub), :],
                    sendL.at[h + 1, s], recvL.at[h + 1, s], left,
                )
                c.start()
                l_next.append(c)
            gemm_store(commR[h], (my + N_DEV - 1 - h) % N_DEV, 0, half)
            gemm_store(commL[h], (my + 1 + h) % N_DEV, half, half)
            r_fly, l_fly = r_next, l_next

        H = N_DEV - 2
        origin_r = (my + 1) % N_DEV
        origin_l = (my + N_DEV - 1) % N_DEV
        for s in range(SUBS):
            r_fly[s].wait()
            gemm_store(
                commR[H, pl.ds(s * sub, sub), :],
                origin_r, s * sub, sub,
            )
            l_fly[s].wait()
            gemm_store(
                commL[H, pl.ds(s * sub, sub), :],
                origin_l, half + s * sub, sub,
            )

    return pl.pallas_call(
        body,
        out_shape=jax.ShapeDtypeStruct((N_DEV * m_per, n_per), jnp.float32),
        in_specs=[
            pl.BlockSpec(memory_space=pl.ANY),
            pl.BlockSpec(memory_space=pltpu.VMEM),
        ],
        out_specs=pl.BlockSpec(memory_space=pltpu.VMEM),
        scratch_shapes=[
            pltpu.VMEM((2, sub, k), jnp.float32),
            pltpu.VMEM((m_per, k), jnp.bfloat16),
            pltpu.VMEM((N_DEV - 1, half, k), jnp.bfloat16),
            pltpu.VMEM((N_DEV - 1, half, k), jnp.bfloat16),
            pltpu.SemaphoreType.DMA((N_DEV - 1, SUBS)),
            pltpu.SemaphoreType.DMA((N_DEV - 1, SUBS)),
            pltpu.SemaphoreType.DMA((N_DEV - 1, SUBS)),
            pltpu.SemaphoreType.DMA((N_DEV - 1, SUBS)),
            pltpu.SemaphoreType.DMA((2,)),
        ],
        compiler_params=pltpu.CompilerParams(
            collective_id=0,
            vmem_limit_bytes=100 * 1024 * 1024,
        ),
    )(x, w_mat)
